# baseline (speedup 1.0000x reference)
import functools

import jax
import jax.numpy as jnp
from jax import lax
from jax.experimental import pallas as pl
from jax.experimental.pallas import tpu as pltpu
from jax.experimental.pallas import tpu_sc as plsc

N = 10000
NP = 10240
D = 128
NSC = 16
NC = 2
NW = NC * NSC
CH = 128
SB = 8
SK = 10
K = SK * SB
E = 320000
EPAD = K * NW * CH
TRASH = NP - 1
RT = NP // NW
RS = NP // NSC
SUB = 64

_MESH = plsc.VectorSubcoreMesh(core_axis_name="c", subcore_axis_name="s")

f32 = jnp.float32


@functools.partial(
    pl.kernel,
    out_type=jax.ShapeDtypeStruct((NC, NP, 16), f32),
    mesh=_MESH,
    scratch_types=[
        pltpu.MemorySpace.VMEM_SHARED((NP, 16), f32),
        pltpu.VMEM((K, CH), jnp.int32),
        pltpu.VMEM((CH, 16), f32),
    ],
)
def _count(cols_hbm, ones_hbm, z16_hbm, degp, deg, colsv, onesv):
    c = lax.axis_index("c")
    s = lax.axis_index("s")
    wid = c * NSC + s
    pltpu.sync_copy(z16_hbm, deg.at[pl.ds(s * RS, RS)])
    pltpu.sync_copy(ones_hbm, onesv)
    pltpu.sync_copy(cols_hbm.at[wid], colsv)
    plsc.subcore_barrier()

    @pl.loop(0, K)
    def _(j):
        pltpu.sync_copy(onesv, deg.at[colsv.at[j]], add=True)

    plsc.subcore_barrier()
    pltpu.sync_copy(deg.at[pl.ds(s * RS, RS)], degp.at[c, pl.ds(s * RS, RS)])


def _norm_body(d0_ref, d1_ref, x_ref, dm_ref, di_ref, u_ref, v_ref):
    deg = d0_ref[...] + d1_ref[...]
    dm = lax.rsqrt(deg + 1.0)
    di = 1.0 / deg
    dm_ref[...] = dm
    di_ref[...] = di
    x = x_ref[...]
    u_ref[...] = x * dm[:, 0:1]
    v_ref[...] = x * di[:, 0:1]


def _norm(degp, Xp):
    blk16 = pl.BlockSpec((1024, 16), lambda i: (i, 0))
    blkD = pl.BlockSpec((1024, D), lambda i: (i, 0))
    return pl.pallas_call(
        _norm_body,
        grid=(NP // 1024,),
        in_specs=[blk16, blk16, blkD],
        out_specs=[blk16, blk16, blkD, blkD],
        out_shape=(
            jax.ShapeDtypeStruct((NP, 16), f32),
            jax.ShapeDtypeStruct((NP, 16), f32),
            jax.ShapeDtypeStruct((NP, D), f32),
            jax.ShapeDtypeStruct((NP, D), f32),
        ),
    )(degp[0], degp[1], Xp)


@functools.partial(
    pl.kernel,
    out_type=jax.ShapeDtypeStruct((NC, NP, D), f32),
    mesh=_MESH,
    scratch_types=[
        pltpu.MemorySpace.VMEM_SHARED((NP, D), f32),
        pltpu.VMEM((K, CH), jnp.int32),
        pltpu.VMEM((K, CH), jnp.int32),
        pltpu.VMEM((CH, D), f32),
    ],
)
def _spmm(u_hbm, cols_hbm, rows_hbm, z_hbm, p_out, acc, colsv, rowsv, gbuf):
    c = lax.axis_index("c")
    s = lax.axis_index("s")
    wid = c * NSC + s
    pltpu.sync_copy(z_hbm, acc.at[pl.ds(s * RS, RS)])
    pltpu.sync_copy(cols_hbm.at[wid], colsv)
    pltpu.sync_copy(rows_hbm.at[wid], rowsv)
    plsc.subcore_barrier()

    @pl.loop(0, K)
    def _(j):
        pltpu.sync_copy(u_hbm.at[colsv.at[j]], gbuf)
        pltpu.sync_copy(gbuf, acc.at[rowsv.at[j]], add=True)

    plsc.subcore_barrier()
    pltpu.sync_copy(acc.at[pl.ds(s * RS, RS)], p_out.at[c, pl.ds(s * RS, RS)])


def _make_dense(combine):
    @functools.partial(
        pl.kernel,
        out_type=(
            jax.ShapeDtypeStruct((NP, D), f32),
            jax.ShapeDtypeStruct((NP, D), f32),
        ),
        mesh=_MESH,
        scratch_types=[
            pltpu.VMEM((SUB, D), f32),
            pltpu.VMEM((SUB, D), f32),
            pltpu.VMEM((SUB, D), f32),
            pltpu.VMEM((SUB, 16), f32),
        ],
    )
    def dense(p_hbm, e_hbm, n_hbm, o1_out, o2_out, b0, b1, b2, nv):
        c = lax.axis_index("c")
        s = lax.axis_index("s")
        base = (c * NSC + s) * RT
        for m in range(RT // SUB):
            r0 = base + m * SUB
            pltpu.sync_copy(p_hbm.at[0, pl.ds(r0, SUB)], b0)
            pltpu.sync_copy(p_hbm.at[1, pl.ds(r0, SUB)], b1)
            pltpu.sync_copy(e_hbm.at[pl.ds(r0, SUB)], b2)
            pltpu.sync_copy(n_hbm.at[pl.ds(r0, SUB)], nv)

            @pl.loop(0, SUB)
            def _(i):
                nrm = nv[i]
                for q in range(D // 16):
                    sl = pl.ds(q * 16, 16)
                    o1, o2 = combine(b0[i, sl], b1[i, sl], b2[i, sl], nrm)
                    b0[i, sl] = o1
                    b1[i, sl] = o2

            pltpu.sync_copy(b0, o1_out.at[pl.ds(r0, SUB)])
            pltpu.sync_copy(b1, o2_out.at[pl.ds(r0, SUB)])

    return dense


def _gcn_combine(p0, p1, uprev, dm):
    f = (p0 + p1 + uprev) * dm
    return f, f * dm


def _scat_combine(p0, p1, fprev, di):
    fp = 0.5 * fprev + 0.5 * (p0 + p1)
    return fp, fp * di


_gcn_dense = _make_dense(_gcn_combine)
_scat_dense = _make_dense(_scat_combine)


_BLK = 1000


def _lrelu(x):
    return jnp.where(x >= 0, x, 0.01 * x)


def _dot_t(lhs, rhs):
    return lax.dot_general(lhs, rhs, (((1,), (1,)), ((), ())),
                           precision=lax.Precision.HIGHEST,
                           preferred_element_type=f32)


def _tc_body(x_ref, g1_ref, g2_ref, g3_ref, f1_ref, f2_ref, f3_ref, f4_ref,
             a_ref, w1_ref, b1_ref, w2_ref, b2_ref, o_ref):
    x = x_ref[...]
    f1, f2, f3, f4 = f1_ref[...], f2_ref[...], f3_ref[...], f4_ref[...]
    hs = [
        _lrelu(g1_ref[...]),
        _lrelu(g2_ref[...]),
        _lrelu(g3_ref[...]),
        jnp.abs(f1 - f2),
        jnp.abs(f2 - f3),
        jnp.abs(f3 - f4),
    ]
    a1 = a_ref[:, :D]
    a2 = a_ref[:, D:]
    c0 = _dot_t(jnp.maximum(x, 0.0), a1)
    e = jnp.concatenate(
        [c0 + _dot_t(jnp.maximum(h, 0.0), a2) for h in hs], axis=1)
    mx = jnp.max(e, axis=1, keepdims=True)
    w = jnp.exp(e - mx)
    att = w / jnp.sum(w, axis=1, keepdims=True)
    hp = att[:, 0:1] * hs[0]
    for kk in range(1, 6):
        hp = hp + att[:, kk:kk + 1] * hs[kk]
    hp = hp * (1.0 / 6.0)
    o = _lrelu(_dot_t(hp, w1_ref[...]) + b1_ref[...])
    o = _lrelu(_dot_t(o, w2_ref[...]) + b2_ref[...])
    o_ref[...] = o


def _tc_tail(X, g1, g2, g3, fp1, fp2, fp3, fp4, a_r, W1, b1_r, W2, b2_r):
    big = pl.BlockSpec((_BLK, D), lambda i: (i, 0))
    full = lambda shp: pl.BlockSpec(shp, lambda i: tuple(0 for _ in shp))
    return pl.pallas_call(
        _tc_body,
        grid=(N // _BLK,),
        in_specs=[big] * 8 + [full((1, 2 * D)), full((D, D)), full((1, D)),
                              full((D, D)), full((1, D))],
        out_specs=big,
        out_shape=jax.ShapeDtypeStruct((N, D), f32),
    )(X, g1, g2, g3, fp1, fp2, fp3, fp4, a_r, W1, b1_r, W2, b2_r)


def kernel(X, edge_index, a, W1, b1, W2, b2, moment):
    rows = edge_index[0].astype(jnp.int32)
    cols = edge_index[1].astype(jnp.int32)
    pad = jnp.full((EPAD - E,), TRASH, jnp.int32)
    cols3 = jnp.concatenate([cols, pad]).reshape(NW, K, CH)
    rows3 = jnp.concatenate([rows, pad]).reshape(NW, K, CH)

    Xp = jnp.pad(X, ((0, NP - N), (0, 0)))
    ones16 = jnp.ones((CH, 16), f32)
    z16 = jnp.zeros((RS, 16), f32)
    zD = jnp.zeros((RS, D), f32)

    degp = _count(cols3, ones16, z16)
    dm16, di16, u, v = _norm(degp, Xp)

    gcns = []
    for _ in range(3):
        p = _spmm(u, cols3, rows3, zD)
        f, u = _gcn_dense(p, u, dm16)
        gcns.append(f)

    fps = []
    fprev = Xp
    for _ in range(4):
        p = _spmm(v, cols3, rows3, zD)
        fprev, v = _scat_dense(p, fprev, di16)
        fps.append(fprev)

    return _tc_tail(X, gcns[0], gcns[1], gcns[2],
                    fps[0], fps[1], fps[2], fps[3],
                    a.reshape(1, 2 * D), W1, b1.reshape(1, D),
                    W2, b2.reshape(1, D))


# speedup vs baseline: 2.6397x; 2.6397x over previous
import functools

import jax
import jax.numpy as jnp
from jax import lax
from jax.experimental import pallas as pl
from jax.experimental.pallas import tpu as pltpu
from jax.experimental.pallas import tpu_sc as plsc

N = 10000
NP = 10240
D = 128
NSC = 16
NC = 2
NW = NC * NSC
CH = 128
SB = 8
SK = 10
K = SK * SB
E = 320000
EPAD = K * NW * CH
TRASH = NP - 1
RT = NP // NW
RS = NP // NSC
SUB = 64

_MESH = plsc.VectorSubcoreMesh(core_axis_name="c", subcore_axis_name="s")

f32 = jnp.float32


@functools.partial(
    pl.kernel,
    out_type=jax.ShapeDtypeStruct((NC, NP, 16), f32),
    mesh=_MESH,
    scratch_types=[
        pltpu.MemorySpace.VMEM_SHARED((NP, 16), f32),
        pltpu.VMEM((K, CH), jnp.int32),
        pltpu.VMEM((CH, 16), f32),
    ],
)
def _count(cols_hbm, ones_hbm, z16_hbm, degp, deg, colsv, onesv):
    c = lax.axis_index("c")
    s = lax.axis_index("s")
    wid = c * NSC + s
    pltpu.sync_copy(z16_hbm, deg.at[pl.ds(s * RS, RS)])
    pltpu.sync_copy(ones_hbm, onesv)
    pltpu.sync_copy(cols_hbm.at[wid], colsv)
    plsc.subcore_barrier()

    @pl.loop(0, K)
    def _(j):
        pltpu.sync_copy(onesv, deg.at[colsv.at[j]], add=True)

    plsc.subcore_barrier()
    pltpu.sync_copy(deg.at[pl.ds(s * RS, RS)], degp.at[c, pl.ds(s * RS, RS)])


def _norm_body(d0_ref, d1_ref, x_ref, dm_ref, di_ref, u_ref, v_ref):
    deg = d0_ref[...] + d1_ref[...]
    dm = lax.rsqrt(deg + 1.0)
    di = 1.0 / deg
    dm_ref[...] = dm
    di_ref[...] = di
    x = x_ref[...]
    u_ref[...] = x * dm[:, 0:1]
    v_ref[...] = x * di[:, 0:1]


def _norm(degp, Xp):
    blk16 = pl.BlockSpec((1024, 16), lambda i: (i, 0))
    blkD = pl.BlockSpec((1024, D), lambda i: (i, 0))
    return pl.pallas_call(
        _norm_body,
        grid=(NP // 1024,),
        in_specs=[blk16, blk16, blkD],
        out_specs=[blk16, blk16, blkD, blkD],
        out_shape=(
            jax.ShapeDtypeStruct((NP, 16), f32),
            jax.ShapeDtypeStruct((NP, 16), f32),
            jax.ShapeDtypeStruct((NP, D), f32),
            jax.ShapeDtypeStruct((NP, D), f32),
        ),
    )(degp[0], degp[1], Xp)


@functools.partial(
    pl.kernel,
    out_type=jax.ShapeDtypeStruct((NC, NP, D), f32),
    mesh=_MESH,
    scratch_types=[
        pltpu.MemorySpace.VMEM_SHARED((NP, D), f32),
        pltpu.VMEM((K, CH), jnp.int32),
        pltpu.VMEM((K, CH), jnp.int32),
        pltpu.VMEM((CH, D), f32),
    ],
)
def _spmm(u_hbm, cols_hbm, rows_hbm, z_hbm, p_out, acc, colsv, rowsv, gbuf):
    c = lax.axis_index("c")
    s = lax.axis_index("s")
    wid = c * NSC + s
    pltpu.sync_copy(z_hbm, acc.at[pl.ds(s * RS, RS)])
    pltpu.sync_copy(cols_hbm.at[wid], colsv)
    pltpu.sync_copy(rows_hbm.at[wid], rowsv)
    plsc.subcore_barrier()

    @pl.loop(0, K)
    def _(j):
        pltpu.sync_copy(u_hbm.at[colsv.at[j]], gbuf)
        pltpu.sync_copy(gbuf, acc.at[rowsv.at[j]], add=True)

    plsc.subcore_barrier()
    pltpu.sync_copy(acc.at[pl.ds(s * RS, RS)], p_out.at[c, pl.ds(s * RS, RS)])


def _make_dense(combine):
    @functools.partial(
        pl.kernel,
        out_type=(
            jax.ShapeDtypeStruct((NP, D), f32),
            jax.ShapeDtypeStruct((NP, D), f32),
        ),
        mesh=_MESH,
        scratch_types=[
            pltpu.VMEM((SUB, D), f32),
            pltpu.VMEM((SUB, D), f32),
            pltpu.VMEM((SUB, D), f32),
            pltpu.VMEM((SUB, 16), f32),
        ],
    )
    def dense(p_hbm, e_hbm, n_hbm, o1_out, o2_out, b0, b1, b2, nv):
        c = lax.axis_index("c")
        s = lax.axis_index("s")
        base = (c * NSC + s) * RT
        for m in range(RT // SUB):
            r0 = base + m * SUB
            pltpu.sync_copy(p_hbm.at[0, pl.ds(r0, SUB)], b0)
            pltpu.sync_copy(p_hbm.at[1, pl.ds(r0, SUB)], b1)
            pltpu.sync_copy(e_hbm.at[pl.ds(r0, SUB)], b2)
            pltpu.sync_copy(n_hbm.at[pl.ds(r0, SUB)], nv)

            @pl.loop(0, SUB)
            def _(i):
                nrm = nv[i]
                for q in range(D // 16):
                    sl = pl.ds(q * 16, 16)
                    o1, o2 = combine(b0[i, sl], b1[i, sl], b2[i, sl], nrm)
                    b0[i, sl] = o1
                    b1[i, sl] = o2

            pltpu.sync_copy(b0, o1_out.at[pl.ds(r0, SUB)])
            pltpu.sync_copy(b1, o2_out.at[pl.ds(r0, SUB)])

    return dense


def _gcn_combine(p0, p1, uprev, dm):
    f = (p0 + p1 + uprev) * dm
    return f, f * dm


def _scat_combine(p0, p1, fprev, di):
    fp = 0.5 * fprev + 0.5 * (p0 + p1)
    return fp, fp * di


_gcn_dense = _make_dense(_gcn_combine)
_scat_dense = _make_dense(_scat_combine)


_BLK = 1000


def _lrelu(x):
    return jnp.where(x >= 0, x, 0.01 * x)


def _dot_t(lhs, rhs):
    return lax.dot_general(lhs, rhs, (((1,), (1,)), ((), ())),
                           precision=lax.Precision.HIGHEST,
                           preferred_element_type=f32)


def _tc_body(x_ref, g1_ref, g2_ref, g3_ref, f1_ref, f2_ref, f3_ref, f4_ref,
             a_ref, w1_ref, b1_ref, w2_ref, b2_ref, o_ref):
    x = x_ref[...]
    f1, f2, f3, f4 = f1_ref[...], f2_ref[...], f3_ref[...], f4_ref[...]
    hs = [
        _lrelu(g1_ref[...]),
        _lrelu(g2_ref[...]),
        _lrelu(g3_ref[...]),
        jnp.abs(f1 - f2),
        jnp.abs(f2 - f3),
        jnp.abs(f3 - f4),
    ]
    a1 = a_ref[:, :D]
    a2 = a_ref[:, D:]
    c0 = _dot_t(jnp.maximum(x, 0.0), a1)
    e = jnp.concatenate(
        [c0 + _dot_t(jnp.maximum(h, 0.0), a2) for h in hs], axis=1)
    mx = jnp.max(e, axis=1, keepdims=True)
    w = jnp.exp(e - mx)
    att = w / jnp.sum(w, axis=1, keepdims=True)
    hp = att[:, 0:1] * hs[0]
    for kk in range(1, 6):
        hp = hp + att[:, kk:kk + 1] * hs[kk]
    hp = hp * (1.0 / 6.0)
    o = _lrelu(_dot_t(hp, w1_ref[...]) + b1_ref[...])
    o = _lrelu(_dot_t(o, w2_ref[...]) + b2_ref[...])
    o_ref[...] = o


def _tc_tail(X, g1, g2, g3, fp1, fp2, fp3, fp4, a_r, W1, b1_r, W2, b2_r):
    big = pl.BlockSpec((_BLK, D), lambda i: (i, 0))
    full = lambda shp: pl.BlockSpec(shp, lambda i: tuple(0 for _ in shp))
    return pl.pallas_call(
        _tc_body,
        grid=(N // _BLK,),
        in_specs=[big] * 8 + [full((1, 2 * D)), full((D, D)), full((1, D)),
                              full((D, D)), full((1, D))],
        out_specs=big,
        out_shape=jax.ShapeDtypeStruct((N, D), f32),
    )(X, g1, g2, g3, fp1, fp2, fp3, fp4, a_r, W1, b1_r, W2, b2_r)


def kernel(X, edge_index, a, W1, b1, W2, b2, moment):
    rows = edge_index[0].astype(jnp.int32)
    cols = edge_index[1].astype(jnp.int32)
    pad = N + jnp.arange(EPAD - E, dtype=jnp.int32) % (NP - N)
    cols3 = jnp.concatenate([cols, pad]).reshape(NW, K, CH)
    rows3 = jnp.concatenate([rows, pad]).reshape(NW, K, CH)

    Xp = jnp.pad(X, ((0, NP - N), (0, 0)))
    ones16 = jnp.ones((CH, 16), f32)
    z16 = jnp.zeros((RS, 16), f32)
    zD = jnp.zeros((RS, D), f32)

    degp = _count(cols3, ones16, z16)
    dm16, di16, u, v = _norm(degp, Xp)

    gcns = []
    for _ in range(3):
        p = _spmm(u, cols3, rows3, zD)
        f, u = _gcn_dense(p, u, dm16)
        gcns.append(f)

    fps = []
    fprev = Xp
    for _ in range(4):
        p = _spmm(v, cols3, rows3, zD)
        fprev, v = _scat_dense(p, fprev, di16)
        fps.append(fprev)

    return _tc_tail(X, gcns[0], gcns[1], gcns[2],
                    fps[0], fps[1], fps[2], fps[3],
                    a.reshape(1, 2 * D), W1, b1.reshape(1, D),
                    W2, b2.reshape(1, D))


# speedup vs baseline: 3.2916x; 1.2470x over previous
import functools

import jax
import jax.numpy as jnp
from jax import lax
from jax.experimental import pallas as pl
from jax.experimental.pallas import tpu as pltpu
from jax.experimental.pallas import tpu_sc as plsc

N = 10000
NP = 10240
D = 128
NSC = 16
NC = 2
NW = NC * NSC
CH = 128
SB = 8
SK = 10
K = SK * SB
E = 320000
EPAD = K * NW * CH
TRASH = NP - 1
RT = NP // NW
RS = NP // NSC
SUB = 64

_MESH = plsc.VectorSubcoreMesh(core_axis_name="c", subcore_axis_name="s")

f32 = jnp.float32


@functools.partial(
    pl.kernel,
    out_type=jax.ShapeDtypeStruct((NC, NP, 16), f32),
    mesh=_MESH,
    scratch_types=[
        pltpu.MemorySpace.VMEM_SHARED((NP, 16), f32),
        pltpu.VMEM((K, CH), jnp.int32),
        pltpu.VMEM((CH, 16), f32),
    ],
)
def _count(cols_hbm, ones_hbm, z16_hbm, degp, deg, colsv, onesv):
    c = lax.axis_index("c")
    s = lax.axis_index("s")
    wid = c * NSC + s
    pltpu.sync_copy(z16_hbm, deg.at[pl.ds(s * RS, RS)])
    pltpu.sync_copy(ones_hbm, onesv)
    pltpu.sync_copy(cols_hbm.at[wid], colsv)
    plsc.subcore_barrier()

    @pl.loop(0, K)
    def _(j):
        pltpu.sync_copy(onesv, deg.at[colsv.at[j]], add=True)

    plsc.subcore_barrier()
    pltpu.sync_copy(deg.at[pl.ds(s * RS, RS)], degp.at[c, pl.ds(s * RS, RS)])


def _norm_body(d0_ref, d1_ref, x_ref, dm_ref, di_ref, u_ref, v_ref):
    deg = d0_ref[...] + d1_ref[...]
    dm = lax.rsqrt(deg + 1.0)
    di = 1.0 / deg
    dm_ref[...] = dm
    di_ref[...] = di
    x = x_ref[...]
    u_ref[...] = x * dm[:, 0:1]
    v_ref[...] = x * di[:, 0:1]


def _norm(degp, Xp):
    blk16 = pl.BlockSpec((1024, 16), lambda i: (i, 0))
    blkD = pl.BlockSpec((1024, D), lambda i: (i, 0))
    return pl.pallas_call(
        _norm_body,
        grid=(NP // 1024,),
        in_specs=[blk16, blk16, blkD],
        out_specs=[blk16, blk16, blkD, blkD],
        out_shape=(
            jax.ShapeDtypeStruct((NP, 16), f32),
            jax.ShapeDtypeStruct((NP, 16), f32),
            jax.ShapeDtypeStruct((NP, D), f32),
            jax.ShapeDtypeStruct((NP, D), f32),
        ),
    )(degp[0], degp[1], Xp)


@functools.partial(
    pl.kernel,
    out_type=jax.ShapeDtypeStruct((NC, NP, D), f32),
    mesh=_MESH,
    scratch_types=[
        pltpu.MemorySpace.VMEM_SHARED((NP, D), f32),
        pltpu.VMEM((SB, CH), jnp.int32),
        pltpu.VMEM((SB, CH), jnp.int32),
        pltpu.VMEM((SB, CH), jnp.int32),
        pltpu.VMEM((SB, CH), jnp.int32),
        pltpu.VMEM((CH, D), f32),
        pltpu.VMEM((CH, D), f32),
        pltpu.SemaphoreType.DMA,
        pltpu.SemaphoreType.DMA,
        pltpu.SemaphoreType.DMA,
        pltpu.SemaphoreType.DMA,
    ],
)
def _spmm(u_hbm, cols_hbm, rows_hbm, z_hbm, p_out,
          acc, cb0, rb0, cb1, rb1, gb0, gb1, sg0, sg1, si0, si1):
    c = lax.axis_index("c")
    s = lax.axis_index("s")
    wid = c * NSC + s
    pltpu.sync_copy(z_hbm, acc.at[pl.ds(s * RS, RS)])
    cbs, rbs, sis = (cb0, cb1), (rb0, rb1), (si0, si1)
    gbs, sgs = (gb0, gb1), (sg0, sg1)

    def start_idx(sc, x):
        pltpu.async_copy(cols_hbm.at[wid, sc], cbs[x], sis[x])
        pltpu.async_copy(rows_hbm.at[wid, sc], rbs[x], sis[x])

    def drain_idx(x):
        d = pltpu.make_async_copy(cols_hbm.at[wid, 0], cbs[x], sis[x])
        d.wait()
        d.wait()

    def start_g(cref, b, g):
        pltpu.async_copy(u_hbm.at[cref.at[b]], gbs[g], sgs[g])

    def drain_g(g):
        pltpu.make_async_copy(u_hbm.at[cb0.at[0]], gbs[g], sgs[g]).wait()

    def do_super(sc, x, last):
        cb, rb = cbs[x], rbs[x]
        for b in range(SB):
            g = b % 2
            drain_g(g)
            if b < SB - 1:
                start_g(cb, b + 1, 1 - g)
            elif not last:
                drain_idx(1 - x)
                start_g(cbs[1 - x], 0, 1 - g)
            pltpu.sync_copy(gbs[g], acc.at[rb.at[b]], add=True)

    pltpu.sync_copy(cols_hbm.at[wid, 0], cb0)
    pltpu.sync_copy(rows_hbm.at[wid, 0], rb0)
    plsc.subcore_barrier()
    start_g(cb0, 0, 0)
    start_idx(1, 1)

    @pl.loop(0, SK // 2 - 1)
    def _(i):
        sc = 2 * i
        do_super(sc, 0, last=False)
        start_idx(sc + 2, 0)
        do_super(sc + 1, 1, last=False)
        start_idx(sc + 3, 1)

    do_super(SK - 2, 0, last=False)
    do_super(SK - 1, 1, last=True)

    plsc.subcore_barrier()
    pltpu.sync_copy(acc.at[pl.ds(s * RS, RS)], p_out.at[c, pl.ds(s * RS, RS)])


def _make_dense(combine):
    @functools.partial(
        pl.kernel,
        out_type=(
            jax.ShapeDtypeStruct((NP, D), f32),
            jax.ShapeDtypeStruct((NP, D), f32),
        ),
        mesh=_MESH,
        scratch_types=[
            pltpu.VMEM((SUB, D), f32),
            pltpu.VMEM((SUB, D), f32),
            pltpu.VMEM((SUB, D), f32),
            pltpu.VMEM((SUB, 16), f32),
        ],
    )
    def dense(p_hbm, e_hbm, n_hbm, o1_out, o2_out, b0, b1, b2, nv):
        c = lax.axis_index("c")
        s = lax.axis_index("s")
        base = (c * NSC + s) * RT
        for m in range(RT // SUB):
            r0 = base + m * SUB
            pltpu.sync_copy(p_hbm.at[0, pl.ds(r0, SUB)], b0)
            pltpu.sync_copy(p_hbm.at[1, pl.ds(r0, SUB)], b1)
            pltpu.sync_copy(e_hbm.at[pl.ds(r0, SUB)], b2)
            pltpu.sync_copy(n_hbm.at[pl.ds(r0, SUB)], nv)

            @pl.loop(0, SUB)
            def _(i):
                nrm = nv[i]
                for q in range(D // 16):
                    sl = pl.ds(q * 16, 16)
                    o1, o2 = combine(b0[i, sl], b1[i, sl], b2[i, sl], nrm)
                    b0[i, sl] = o1
                    b1[i, sl] = o2

            pltpu.sync_copy(b0, o1_out.at[pl.ds(r0, SUB)])
            pltpu.sync_copy(b1, o2_out.at[pl.ds(r0, SUB)])

    return dense


def _gcn_combine(p0, p1, uprev, dm):
    f = (p0 + p1 + uprev) * dm
    return f, f * dm


def _scat_combine(p0, p1, fprev, di):
    fp = 0.5 * fprev + 0.5 * (p0 + p1)
    return fp, fp * di


_gcn_dense = _make_dense(_gcn_combine)
_scat_dense = _make_dense(_scat_combine)


_BLK = 1000


def _lrelu(x):
    return jnp.where(x >= 0, x, 0.01 * x)


def _dot_t(lhs, rhs):
    return lax.dot_general(lhs, rhs, (((1,), (1,)), ((), ())),
                           precision=lax.Precision.HIGHEST,
                           preferred_element_type=f32)


def _tc_body(x_ref, g1_ref, g2_ref, g3_ref, f1_ref, f2_ref, f3_ref, f4_ref,
             a_ref, w1_ref, b1_ref, w2_ref, b2_ref, o_ref):
    x = x_ref[...]
    f1, f2, f3, f4 = f1_ref[...], f2_ref[...], f3_ref[...], f4_ref[...]
    hs = [
        _lrelu(g1_ref[...]),
        _lrelu(g2_ref[...]),
        _lrelu(g3_ref[...]),
        jnp.abs(f1 - f2),
        jnp.abs(f2 - f3),
        jnp.abs(f3 - f4),
    ]
    a1 = a_ref[:, :D]
    a2 = a_ref[:, D:]
    c0 = _dot_t(jnp.maximum(x, 0.0), a1)
    e = jnp.concatenate(
        [c0 + _dot_t(jnp.maximum(h, 0.0), a2) for h in hs], axis=1)
    mx = jnp.max(e, axis=1, keepdims=True)
    w = jnp.exp(e - mx)
    att = w / jnp.sum(w, axis=1, keepdims=True)
    hp = att[:, 0:1] * hs[0]
    for kk in range(1, 6):
        hp = hp + att[:, kk:kk + 1] * hs[kk]
    hp = hp * (1.0 / 6.0)
    o = _lrelu(_dot_t(hp, w1_ref[...]) + b1_ref[...])
    o = _lrelu(_dot_t(o, w2_ref[...]) + b2_ref[...])
    o_ref[...] = o


def _tc_tail(X, g1, g2, g3, fp1, fp2, fp3, fp4, a_r, W1, b1_r, W2, b2_r):
    big = pl.BlockSpec((_BLK, D), lambda i: (i, 0))
    full = lambda shp: pl.BlockSpec(shp, lambda i: tuple(0 for _ in shp))
    return pl.pallas_call(
        _tc_body,
        grid=(N // _BLK,),
        in_specs=[big] * 8 + [full((1, 2 * D)), full((D, D)), full((1, D)),
                              full((D, D)), full((1, D))],
        out_specs=big,
        out_shape=jax.ShapeDtypeStruct((N, D), f32),
    )(X, g1, g2, g3, fp1, fp2, fp3, fp4, a_r, W1, b1_r, W2, b2_r)


def kernel(X, edge_index, a, W1, b1, W2, b2, moment):
    rows = edge_index[0].astype(jnp.int32)
    cols = edge_index[1].astype(jnp.int32)
    pad = N + jnp.arange(EPAD - E, dtype=jnp.int32) % (NP - N)
    colsp = jnp.concatenate([cols, pad])
    rowsp = jnp.concatenate([rows, pad])
    cols3 = colsp.reshape(NW, K, CH)
    cols4 = colsp.reshape(NW, SK, SB, CH)
    rows4 = rowsp.reshape(NW, SK, SB, CH)

    Xp = jnp.pad(X, ((0, NP - N), (0, 0)))
    ones16 = jnp.ones((CH, 16), f32)
    z16 = jnp.zeros((RS, 16), f32)
    zD = jnp.zeros((RS, D), f32)

    degp = _count(cols3, ones16, z16)
    dm16, di16, u, v = _norm(degp, Xp)

    gcns = []
    for _ in range(3):
        p = _spmm(u, cols4, rows4, zD)
        f, u = _gcn_dense(p, u, dm16)
        gcns.append(f)

    fps = []
    fprev = Xp
    for _ in range(4):
        p = _spmm(v, cols4, rows4, zD)
        fprev, v = _scat_dense(p, fprev, di16)
        fps.append(fprev)

    return _tc_tail(X, gcns[0], gcns[1], gcns[2],
                    fps[0], fps[1], fps[2], fps[3],
                    a.reshape(1, 2 * D), W1, b1.reshape(1, D),
                    W2, b2.reshape(1, D))


# speedup vs baseline: 3.3555x; 1.0194x over previous
import functools

import jax
import jax.numpy as jnp
from jax import lax
from jax.experimental import pallas as pl
from jax.experimental.pallas import tpu as pltpu
from jax.experimental.pallas import tpu_sc as plsc

N = 10000
NP = 10240
D = 128
NSC = 16
NC = 2
NW = NC * NSC
CH = 128
SB = 8
SK = 10
K = SK * SB
E = 320000
EPAD = K * NW * CH
TRASH = NP - 1
RT = NP // NW
RS = NP // NSC
SUB = 64

_MESH = plsc.VectorSubcoreMesh(core_axis_name="c", subcore_axis_name="s")

f32 = jnp.float32


@functools.partial(
    pl.kernel,
    out_type=jax.ShapeDtypeStruct((NC, NP, 16), f32),
    mesh=_MESH,
    scratch_types=[
        pltpu.MemorySpace.VMEM_SHARED((NP, 16), f32),
        pltpu.VMEM((K, CH), jnp.int32),
        pltpu.VMEM((CH, 16), f32),
    ],
)
def _count(cols_hbm, ones_hbm, z16_hbm, degp, deg, colsv, onesv):
    c = lax.axis_index("c")
    s = lax.axis_index("s")
    wid = c * NSC + s
    pltpu.sync_copy(z16_hbm, deg.at[pl.ds(s * RS, RS)])
    pltpu.sync_copy(ones_hbm, onesv)
    pltpu.sync_copy(cols_hbm.at[wid], colsv)
    plsc.subcore_barrier()

    @pl.loop(0, K)
    def _(j):
        pltpu.sync_copy(onesv, deg.at[colsv.at[j]], add=True)

    plsc.subcore_barrier()
    pltpu.sync_copy(deg.at[pl.ds(s * RS, RS)], degp.at[c, pl.ds(s * RS, RS)])


def _norm_body(d0_ref, d1_ref, x_ref, dm_ref, di_ref, u_ref, v_ref):
    deg = d0_ref[...] + d1_ref[...]
    dm = lax.rsqrt(deg + 1.0)
    di = 1.0 / deg
    dm_ref[...] = dm
    di_ref[...] = di
    x = x_ref[...]
    u_ref[...] = x * dm[:, 0:1]
    v_ref[...] = x * di[:, 0:1]


def _norm(degp, Xp):
    blk16 = pl.BlockSpec((1024, 16), lambda i: (i, 0))
    blkD = pl.BlockSpec((1024, D), lambda i: (i, 0))
    return pl.pallas_call(
        _norm_body,
        grid=(NP // 1024,),
        in_specs=[blk16, blk16, blkD],
        out_specs=[blk16, blk16, blkD, blkD],
        out_shape=(
            jax.ShapeDtypeStruct((NP, 16), f32),
            jax.ShapeDtypeStruct((NP, 16), f32),
            jax.ShapeDtypeStruct((NP, D), f32),
            jax.ShapeDtypeStruct((NP, D), f32),
        ),
    )(degp[0], degp[1], Xp)


@functools.partial(
    pl.kernel,
    out_type=jax.ShapeDtypeStruct((NC, NP, D), f32),
    mesh=_MESH,
    scratch_types=[
        pltpu.MemorySpace.VMEM_SHARED((NP, D), f32),
        pltpu.VMEM((SB, CH), jnp.int32),
        pltpu.VMEM((SB, CH), jnp.int32),
        pltpu.VMEM((SB, CH), jnp.int32),
        pltpu.VMEM((SB, CH), jnp.int32),
        pltpu.VMEM((CH, D), f32),
        pltpu.VMEM((CH, D), f32),
        pltpu.SemaphoreType.DMA,
        pltpu.SemaphoreType.DMA,
        pltpu.SemaphoreType.DMA,
        pltpu.SemaphoreType.DMA,
    ],
)
def _spmm(u_hbm, cols_hbm, rows_hbm, z_hbm, p_out,
          acc, cb0, rb0, cb1, rb1, gb0, gb1, sg0, sg1, si0, si1):
    c = lax.axis_index("c")
    s = lax.axis_index("s")
    wid = c * NSC + s
    pltpu.sync_copy(z_hbm, acc.at[pl.ds(s * RS, RS)])
    cbs, rbs, sis = (cb0, cb1), (rb0, rb1), (si0, si1)
    gbs, sgs = (gb0, gb1), (sg0, sg1)

    def start_idx(sc, x):
        pltpu.async_copy(cols_hbm.at[wid, sc], cbs[x], sis[x])
        pltpu.async_copy(rows_hbm.at[wid, sc], rbs[x], sis[x])

    def drain_idx(x):
        d = pltpu.make_async_copy(cols_hbm.at[wid, 0], cbs[x], sis[x])
        d.wait()
        d.wait()

    def start_g(cref, b, g):
        pltpu.async_copy(u_hbm.at[cref.at[b]], gbs[g], sgs[g])

    def drain_g(g):
        pltpu.make_async_copy(u_hbm.at[cb0.at[0]], gbs[g], sgs[g]).wait()

    def do_super(sc, x, last):
        cb, rb = cbs[x], rbs[x]
        for b in range(SB):
            g = b % 2
            drain_g(g)
            if b < SB - 1:
                start_g(cb, b + 1, 1 - g)
            elif not last:
                drain_idx(1 - x)
                start_g(cbs[1 - x], 0, 1 - g)
            pltpu.sync_copy(gbs[g], acc.at[rb.at[b]], add=True)

    pltpu.sync_copy(cols_hbm.at[wid, 0], cb0)
    pltpu.sync_copy(rows_hbm.at[wid, 0], rb0)
    plsc.subcore_barrier()
    start_g(cb0, 0, 0)
    start_idx(1, 1)

    @pl.loop(0, SK // 2 - 1)
    def _(i):
        sc = 2 * i
        do_super(sc, 0, last=False)
        start_idx(sc + 2, 0)
        do_super(sc + 1, 1, last=False)
        start_idx(sc + 3, 1)

    do_super(SK - 2, 0, last=False)
    do_super(SK - 1, 1, last=True)

    plsc.subcore_barrier()
    pltpu.sync_copy(acc.at[pl.ds(s * RS, RS)], p_out.at[c, pl.ds(s * RS, RS)])


def _make_dense(combine):
    @functools.partial(
        pl.kernel,
        out_type=(
            jax.ShapeDtypeStruct((NP, D), f32),
            jax.ShapeDtypeStruct((NP, D), f32),
        ),
        mesh=_MESH,
        scratch_types=[
            pltpu.VMEM((RT, D), f32),
            pltpu.VMEM((RT, D), f32),
            pltpu.VMEM((RT, D), f32),
            pltpu.VMEM((RT // 8, D), f32),
        ],
    )
    def dense(p_hbm, e_hbm, n_hbm, o1_out, o2_out, b0, b1, b2, nv):
        c = lax.axis_index("c")
        s = lax.axis_index("s")
        r0 = (c * NSC + s) * RT
        pltpu.sync_copy(p_hbm.at[0, pl.ds(r0, RT)], b0)
        pltpu.sync_copy(p_hbm.at[1, pl.ds(r0, RT)], b1)
        pltpu.sync_copy(e_hbm.at[pl.ds(r0, RT)], b2)
        pltpu.sync_copy(n_hbm.at[pl.ds((c * NSC + s) * (RT // 8), RT // 8)], nv)

        @pl.loop(0, RT, unroll=2)
        def _(i):
            nrm = nv[i >> 3, pl.ds((i & 7) * 16, 16)]
            for q in range(D // 16):
                sl = pl.ds(q * 16, 16)
                o1, o2 = combine(b0[i, sl], b1[i, sl], b2[i, sl], nrm)
                b0[i, sl] = o1
                b1[i, sl] = o2

        pltpu.sync_copy(b0, o1_out.at[pl.ds(r0, RT)])
        pltpu.sync_copy(b1, o2_out.at[pl.ds(r0, RT)])

    return dense


def _gcn_combine(p0, p1, uprev, dm):
    f = (p0 + p1 + uprev) * dm
    return f, f * dm


def _scat_combine(p0, p1, fprev, di):
    fp = 0.5 * fprev + 0.5 * (p0 + p1)
    return fp, fp * di


_gcn_dense = _make_dense(_gcn_combine)
_scat_dense = _make_dense(_scat_combine)


_BLK = 1000


def _lrelu(x):
    return jnp.where(x >= 0, x, 0.01 * x)


def _dot_t(lhs, rhs):
    return lax.dot_general(lhs, rhs, (((1,), (1,)), ((), ())),
                           precision=lax.Precision.HIGHEST,
                           preferred_element_type=f32)


def _tc_body(x_ref, g1_ref, g2_ref, p30_ref, p31_ref, u2_ref, dm_ref,
             f1_ref, f2_ref, f3_ref, q40_ref, q41_ref,
             a_ref, w1_ref, b1_ref, w2_ref, b2_ref, o_ref):
    x = x_ref[...]
    f1, f2, f3 = f1_ref[...], f2_ref[...], f3_ref[...]
    g3 = (p30_ref[...] + p31_ref[...] + u2_ref[...]) * dm_ref[...][:, 0:1]
    f4 = 0.5 * f3 + 0.5 * (q40_ref[...] + q41_ref[...])
    hs = [
        _lrelu(g1_ref[...]),
        _lrelu(g2_ref[...]),
        _lrelu(g3),
        jnp.abs(f1 - f2),
        jnp.abs(f2 - f3),
        jnp.abs(f3 - f4),
    ]
    a1 = a_ref[:, :D]
    a2 = a_ref[:, D:]
    c0 = _dot_t(jnp.maximum(x, 0.0), a1)
    e = jnp.concatenate(
        [c0 + _dot_t(jnp.maximum(h, 0.0), a2) for h in hs], axis=1)
    mx = jnp.max(e, axis=1, keepdims=True)
    w = jnp.exp(e - mx)
    att = w / jnp.sum(w, axis=1, keepdims=True)
    hp = att[:, 0:1] * hs[0]
    for kk in range(1, 6):
        hp = hp + att[:, kk:kk + 1] * hs[kk]
    hp = hp * (1.0 / 6.0)
    o = _lrelu(_dot_t(hp, w1_ref[...]) + b1_ref[...])
    o = _lrelu(_dot_t(o, w2_ref[...]) + b2_ref[...])
    o_ref[...] = o


def _tc_tail(X, g1, g2, p3, u2, dm16, fp1, fp2, fp3, q4,
             a_r, W1, b1_r, W2, b2_r):
    big = pl.BlockSpec((_BLK, D), lambda i: (i, 0))
    blk16 = pl.BlockSpec((_BLK, 16), lambda i: (i, 0))
    full = lambda shp: pl.BlockSpec(shp, lambda i: tuple(0 for _ in shp))
    return pl.pallas_call(
        _tc_body,
        grid=(N // _BLK,),
        in_specs=[big] * 6 + [blk16] + [big] * 5
        + [full((1, 2 * D)), full((D, D)), full((1, D)),
           full((D, D)), full((1, D))],
        out_specs=big,
        out_shape=jax.ShapeDtypeStruct((N, D), f32),
    )(X, g1, g2, p3[0], p3[1], u2, dm16, fp1, fp2, fp3, q4[0], q4[1],
      a_r, W1, b1_r, W2, b2_r)


def kernel(X, edge_index, a, W1, b1, W2, b2, moment):
    rows = edge_index[0].astype(jnp.int32)
    cols = edge_index[1].astype(jnp.int32)
    pad = N + jnp.arange(EPAD - E, dtype=jnp.int32) % (NP - N)
    colsp = jnp.concatenate([cols, pad])
    rowsp = jnp.concatenate([rows, pad])
    cols3 = colsp.reshape(NW, K, CH)
    cols4 = colsp.reshape(NW, SK, SB, CH)
    rows4 = rowsp.reshape(NW, SK, SB, CH)

    Xp = jnp.pad(X, ((0, NP - N), (0, 0)))
    ones16 = jnp.ones((CH, 16), f32)
    z16 = jnp.zeros((RS, 16), f32)
    zD = jnp.zeros((RS, D), f32)

    degp = _count(cols3, ones16, z16)
    dm16, di16, u, v = _norm(degp, Xp)

    dmr = dm16.reshape(NP // 8, D)
    dir_ = di16.reshape(NP // 8, D)

    p1 = _spmm(u, cols4, rows4, zD)
    g1, u1 = _gcn_dense(p1, u, dmr)
    p2 = _spmm(u1, cols4, rows4, zD)
    g2, u2 = _gcn_dense(p2, u1, dmr)
    p3 = _spmm(u2, cols4, rows4, zD)

    q1 = _spmm(v, cols4, rows4, zD)
    fp1, v1 = _scat_dense(q1, Xp, dir_)
    q2 = _spmm(v1, cols4, rows4, zD)
    fp2, v2 = _scat_dense(q2, fp1, dir_)
    q3 = _spmm(v2, cols4, rows4, zD)
    fp3, v3 = _scat_dense(q3, fp2, dir_)
    q4 = _spmm(v3, cols4, rows4, zD)

    return _tc_tail(X, g1, g2, p3, u2, dm16, fp1, fp2, fp3, q4,
                    a.reshape(1, 2 * D), W1, b1.reshape(1, D),
                    W2, b2.reshape(1, D))


# speedup vs baseline: 3.4290x; 1.0219x over previous
import functools

import jax
import jax.numpy as jnp
from jax import lax
from jax.experimental import pallas as pl
from jax.experimental.pallas import tpu as pltpu
from jax.experimental.pallas import tpu_sc as plsc

N = 10000
NP = 10240
D = 128
NSC = 16
NC = 2
NW = NC * NSC
CH = 128
SB = 8
SK = 10
K = SK * SB
E = 320000
EPAD = K * NW * CH
TRASH = NP - 1
RT = NP // NW
RS = NP // NSC
SUB = 64

_MESH = plsc.VectorSubcoreMesh(core_axis_name="c", subcore_axis_name="s")

f32 = jnp.float32


@functools.partial(
    pl.kernel,
    out_type=jax.ShapeDtypeStruct((NC, NP, 16), f32),
    mesh=_MESH,
    scratch_types=[
        pltpu.MemorySpace.VMEM_SHARED((NP, 16), f32),
        pltpu.VMEM((K, CH), jnp.int32),
        pltpu.VMEM((CH, 16), f32),
    ],
)
def _count(cols_hbm, ones_hbm, z16_hbm, degp, deg, colsv, onesv):
    c = lax.axis_index("c")
    s = lax.axis_index("s")
    wid = c * NSC + s
    pltpu.sync_copy(z16_hbm, deg.at[pl.ds(s * RS, RS)])
    pltpu.sync_copy(ones_hbm, onesv)
    pltpu.sync_copy(cols_hbm.at[wid], colsv)
    plsc.subcore_barrier()

    @pl.loop(0, K)
    def _(j):
        pltpu.sync_copy(onesv, deg.at[colsv.at[j]], add=True)

    plsc.subcore_barrier()
    pltpu.sync_copy(deg.at[pl.ds(s * RS, RS)], degp.at[c, pl.ds(s * RS, RS)])


def _norm_body(d0_ref, d1_ref, x_ref, dm_ref, di_ref, u_ref, v_ref):
    deg = d0_ref[...] + d1_ref[...]
    dm = lax.rsqrt(deg + 1.0)
    di = 1.0 / deg
    dm_ref[...] = dm
    di_ref[...] = di
    x = x_ref[...]
    u_ref[...] = x * dm[:, 0:1]
    v_ref[...] = x * di[:, 0:1]


def _norm(degp, Xp):
    blk16 = pl.BlockSpec((1024, 16), lambda i: (i, 0))
    blkD = pl.BlockSpec((1024, D), lambda i: (i, 0))
    return pl.pallas_call(
        _norm_body,
        grid=(NP // 1024,),
        in_specs=[blk16, blk16, blkD],
        out_specs=[blk16, blk16, blkD, blkD],
        out_shape=(
            jax.ShapeDtypeStruct((NP, 16), f32),
            jax.ShapeDtypeStruct((NP, 16), f32),
            jax.ShapeDtypeStruct((NP, D), f32),
            jax.ShapeDtypeStruct((NP, D), f32),
        ),
    )(degp[0], degp[1], Xp)


@functools.partial(
    pl.kernel,
    out_type=jax.ShapeDtypeStruct((NC, NP, D), f32),
    mesh=_MESH,
    scratch_types=[
        pltpu.MemorySpace.VMEM_SHARED((NP, D), f32),
        pltpu.VMEM((SB, CH), jnp.int32),
        pltpu.VMEM((SB, CH), jnp.int32),
        pltpu.VMEM((SB, CH), jnp.int32),
        pltpu.VMEM((SB, CH), jnp.int32),
        pltpu.VMEM((CH, D), f32),
        pltpu.VMEM((CH, D), f32),
        pltpu.SemaphoreType.DMA,
        pltpu.SemaphoreType.DMA,
        pltpu.SemaphoreType.DMA,
        pltpu.SemaphoreType.DMA,
    ],
)
def _spmm(u_hbm, cols_hbm, rows_hbm, p_out,
          acc, cb0, rb0, cb1, rb1, gb0, gb1, sg0, sg1, si0, si1):
    c = lax.axis_index("c")
    s = lax.axis_index("s")
    wid = c * NSC + s

    @pl.loop(0, CH)
    def _(i):
        for q in range(D // 16):
            gb0[i, pl.ds(q * 16, 16)] = jnp.zeros((16,), f32)

    for t in range(RS // CH):
        pltpu.sync_copy(gb0, acc.at[pl.ds(s * RS + t * CH, CH)])
    cbs, rbs, sis = (cb0, cb1), (rb0, rb1), (si0, si1)
    gbs, sgs = (gb0, gb1), (sg0, sg1)

    def start_idx(sc, x):
        pltpu.async_copy(cols_hbm.at[wid, sc], cbs[x], sis[x])
        pltpu.async_copy(rows_hbm.at[wid, sc], rbs[x], sis[x])

    def drain_idx(x):
        d = pltpu.make_async_copy(cols_hbm.at[wid, 0], cbs[x], sis[x])
        d.wait()
        d.wait()

    def start_g(cref, b, g):
        pltpu.async_copy(u_hbm.at[cref.at[b]], gbs[g], sgs[g])

    def drain_g(g):
        pltpu.make_async_copy(u_hbm.at[cb0.at[0]], gbs[g], sgs[g]).wait()

    def do_super(sc, x, last):
        cb, rb = cbs[x], rbs[x]
        for b in range(SB):
            g = b % 2
            drain_g(g)
            if b < SB - 1:
                start_g(cb, b + 1, 1 - g)
            elif not last:
                drain_idx(1 - x)
                start_g(cbs[1 - x], 0, 1 - g)
            pltpu.sync_copy(gbs[g], acc.at[rb.at[b]], add=True)

    pltpu.sync_copy(cols_hbm.at[wid, 0], cb0)
    pltpu.sync_copy(rows_hbm.at[wid, 0], rb0)
    plsc.subcore_barrier()
    start_g(cb0, 0, 0)
    start_idx(1, 1)

    @pl.loop(0, SK // 2 - 1)
    def _(i):
        sc = 2 * i
        do_super(sc, 0, last=False)
        start_idx(sc + 2, 0)
        do_super(sc + 1, 1, last=False)
        start_idx(sc + 3, 1)

    do_super(SK - 2, 0, last=False)
    do_super(SK - 1, 1, last=True)

    plsc.subcore_barrier()
    pltpu.sync_copy(acc.at[pl.ds(s * RS, RS)], p_out.at[c, pl.ds(s * RS, RS)])


def _make_dense(combine):
    @functools.partial(
        pl.kernel,
        out_type=(
            jax.ShapeDtypeStruct((NP, D), f32),
            jax.ShapeDtypeStruct((NP, D), f32),
        ),
        mesh=_MESH,
        scratch_types=[
            pltpu.VMEM((RT, D), f32),
            pltpu.VMEM((RT, D), f32),
            pltpu.VMEM((RT, D), f32),
            pltpu.VMEM((RT // 8, D), f32),
        ],
    )
    def dense(p_hbm, e_hbm, n_hbm, o1_out, o2_out, b0, b1, b2, nv):
        c = lax.axis_index("c")
        s = lax.axis_index("s")
        r0 = (c * NSC + s) * RT
        pltpu.sync_copy(p_hbm.at[0, pl.ds(r0, RT)], b0)
        pltpu.sync_copy(p_hbm.at[1, pl.ds(r0, RT)], b1)
        pltpu.sync_copy(e_hbm.at[pl.ds(r0, RT)], b2)
        pltpu.sync_copy(n_hbm.at[pl.ds((c * NSC + s) * (RT // 8), RT // 8)], nv)

        @pl.loop(0, RT, unroll=2)
        def _(i):
            nrm = nv[i >> 3, pl.ds((i & 7) * 16, 16)]
            for q in range(D // 16):
                sl = pl.ds(q * 16, 16)
                o1, o2 = combine(b0[i, sl], b1[i, sl], b2[i, sl], nrm)
                b0[i, sl] = o1
                b1[i, sl] = o2

        pltpu.sync_copy(b0, o1_out.at[pl.ds(r0, RT)])
        pltpu.sync_copy(b1, o2_out.at[pl.ds(r0, RT)])

    return dense


def _gcn_combine(p0, p1, uprev, dm):
    f = (p0 + p1 + uprev) * dm
    return f, f * dm


def _scat_combine(p0, p1, fprev, di):
    fp = 0.5 * fprev + 0.5 * (p0 + p1)
    return fp, fp * di


_gcn_dense = _make_dense(_gcn_combine)
_scat_dense = _make_dense(_scat_combine)


_BLK = 1000


def _lrelu(x):
    return jnp.where(x >= 0, x, 0.01 * x)


def _dot_t(lhs, rhs):
    return lax.dot_general(lhs, rhs, (((1,), (1,)), ((), ())),
                           precision=lax.Precision.HIGHEST,
                           preferred_element_type=f32)


def _tc_body(x_ref, g1_ref, g2_ref, p30_ref, p31_ref, u2_ref, dm_ref,
             f1_ref, f2_ref, f3_ref, q40_ref, q41_ref,
             a_ref, w1_ref, b1_ref, w2_ref, b2_ref, o_ref):
    x = x_ref[...]
    f1, f2, f3 = f1_ref[...], f2_ref[...], f3_ref[...]
    g3 = (p30_ref[...] + p31_ref[...] + u2_ref[...]) * dm_ref[...][:, 0:1]
    f4 = 0.5 * f3 + 0.5 * (q40_ref[...] + q41_ref[...])
    hs = [
        _lrelu(g1_ref[...]),
        _lrelu(g2_ref[...]),
        _lrelu(g3),
        jnp.abs(f1 - f2),
        jnp.abs(f2 - f3),
        jnp.abs(f3 - f4),
    ]
    a1 = a_ref[:, :D]
    a2 = a_ref[:, D:]
    c0 = _dot_t(jnp.maximum(x, 0.0), a1)
    e = jnp.concatenate(
        [c0 + _dot_t(jnp.maximum(h, 0.0), a2) for h in hs], axis=1)
    mx = jnp.max(e, axis=1, keepdims=True)
    w = jnp.exp(e - mx)
    att = w / jnp.sum(w, axis=1, keepdims=True)
    hp = att[:, 0:1] * hs[0]
    for kk in range(1, 6):
        hp = hp + att[:, kk:kk + 1] * hs[kk]
    hp = hp * (1.0 / 6.0)
    o = _lrelu(_dot_t(hp, w1_ref[...]) + b1_ref[...])
    o = _lrelu(_dot_t(o, w2_ref[...]) + b2_ref[...])
    o_ref[...] = o


def _tc_tail(X, g1, g2, p3, u2, dm16, fp1, fp2, fp3, q4,
             a_r, W1, b1_r, W2, b2_r):
    big = pl.BlockSpec((_BLK, D), lambda i: (i, 0))
    blk16 = pl.BlockSpec((_BLK, 16), lambda i: (i, 0))
    full = lambda shp: pl.BlockSpec(shp, lambda i: tuple(0 for _ in shp))
    return pl.pallas_call(
        _tc_body,
        grid=(N // _BLK,),
        in_specs=[big] * 6 + [blk16] + [big] * 5
        + [full((1, 2 * D)), full((D, D)), full((1, D)),
           full((D, D)), full((1, D))],
        out_specs=big,
        out_shape=jax.ShapeDtypeStruct((N, D), f32),
    )(X, g1, g2, p3[0], p3[1], u2, dm16, fp1, fp2, fp3, q4[0], q4[1],
      a_r, W1, b1_r, W2, b2_r)


def kernel(X, edge_index, a, W1, b1, W2, b2, moment):
    rows = edge_index[0].astype(jnp.int32)
    cols = edge_index[1].astype(jnp.int32)
    pad = N + jnp.arange(EPAD - E, dtype=jnp.int32) % (NP - N)
    colsp = jnp.concatenate([cols, pad])
    rowsp = jnp.concatenate([rows, pad])
    cols3 = colsp.reshape(NW, K, CH)
    cols4 = colsp.reshape(NW, SK, SB, CH)
    rows4 = rowsp.reshape(NW, SK, SB, CH)

    Xp = jnp.pad(X, ((0, NP - N), (0, 0)))
    ones16 = jnp.ones((CH, 16), f32)
    z16 = jnp.zeros((RS, 16), f32)

    degp = _count(cols3, ones16, z16)
    dm16, di16, u, v = _norm(degp, Xp)

    dmr = dm16.reshape(NP // 8, D)
    dir_ = di16.reshape(NP // 8, D)

    p1 = _spmm(u, cols4, rows4)
    g1, u1 = _gcn_dense(p1, u, dmr)
    p2 = _spmm(u1, cols4, rows4)
    g2, u2 = _gcn_dense(p2, u1, dmr)
    p3 = _spmm(u2, cols4, rows4)

    q1 = _spmm(v, cols4, rows4)
    fp1, v1 = _scat_dense(q1, Xp, dir_)
    q2 = _spmm(v1, cols4, rows4)
    fp2, v2 = _scat_dense(q2, fp1, dir_)
    q3 = _spmm(v2, cols4, rows4)
    fp3, v3 = _scat_dense(q3, fp2, dir_)
    q4 = _spmm(v3, cols4, rows4)

    return _tc_tail(X, g1, g2, p3, u2, dm16, fp1, fp2, fp3, q4,
                    a.reshape(1, 2 * D), W1, b1.reshape(1, D),
                    W2, b2.reshape(1, D))
